# baseline (speedup 1.0000x reference)
import functools

import ml_dtypes
import numpy as np
import jax
import jax.numpy as jnp
from jax import lax
from jax.experimental import pallas as pl
from jax.experimental.pallas import tpu as pltpu
from jax.experimental.pallas import tpu_sc as plsc

_MAX_LEN = 2048

_NUM_CORES = 2
_NUM_SUBCORES = 16
_NUM_WORKERS = _NUM_CORES * _NUM_SUBCORES
_LANES = 16
_NBUF = 4


def _positional_encoding(max_len, d_model):
    pos = np.arange(max_len, dtype=np.float32)[:, None]
    i2 = np.arange(0, d_model, 2, dtype=np.float32)
    div = np.power(10000.0, i2 / d_model)
    pe = np.zeros((max_len, d_model), dtype=np.float32)
    pe[:, 0::2] = np.sin(pos / div)
    pe[:, 1::2] = np.cos(pos / div)
    return pe


_PE_SCALE = 127.0


def _packed_pe(S, D):
    pe = _positional_encoding(_MAX_LEN, D)[:S]
    q = np.rint(pe * _PE_SCALE).astype(np.int8)
    grp = q.reshape(S, D // 64, 64)
    packed = np.empty_like(grp)
    packed[:, :, 0::4] = grp[:, :, 0:16]
    packed[:, :, 1::4] = grp[:, :, 32:48]
    packed[:, :, 2::4] = grp[:, :, 16:32]
    packed[:, :, 3::4] = grp[:, :, 48:64]
    return jnp.asarray(packed.reshape(S * D // 4, 4).view(np.int32)[:, 0])


@functools.cache
def _build_kernel(B, S, D, C):
    N = B * S
    W = S // _NUM_WORKERS
    n_chunks = B * W // C
    per_b = W // C
    mesh = plsc.VectorSubcoreMesh(core_axis_name="c", subcore_axis_name="s")

    @functools.partial(
        pl.kernel,
        out_type=jax.ShapeDtypeStruct((N, D), jnp.float32),
        mesh=mesh,
        compiler_params=pltpu.CompilerParams(needs_layout_passes=False),
        scratch_types=[
            pltpu.VMEM((B, W), jnp.int32),
            pltpu.VMEM((W * D // 4,), jnp.int32),
            pltpu.VMEM((_NBUF, C, D), jnp.float32),
            [pltpu.SemaphoreType.DMA] * _NBUF,
            [pltpu.SemaphoreType.DMA] * _NBUF,
            pltpu.SemaphoreType.DMA,
            pltpu.SemaphoreType.DMA,
        ],
    )
    def emb_kernel(x_hbm, table_hbm, pe_hbm, out_hbm,
                   idx_v, pe_res, rows_v, gsem, ssem, isem, psem):
        wid = lax.axis_index("s") * _NUM_CORES + lax.axis_index("c")
        p0 = wid * W

        pe_copy = pltpu.async_copy(
            pe_hbm.at[pl.ds(pl.multiple_of(p0 * (D // 4), 8), W * D // 4)],
            pe_res, psem)
        idx_copies = [
            pltpu.async_copy(x_hbm.at[blk, pl.ds(p0, W)], idx_v.at[blk], isem)
            for blk in range(B)
        ]
        idx_waited = [False] * B

        def start_gather(c):
            blk = c // per_b
            if not idx_waited[blk]:
                idx_copies[blk].wait()
                idx_waited[blk] = True
            b = c % _NBUF
            return pltpu.async_copy(
                table_hbm.at[idx_v.at[blk, pl.ds((c % per_b) * C, C)]],
                rows_v.at[b], gsem[b])

        inflight = {}
        stores = {}
        for c in range(min(_NBUF - 1, n_chunks)):
            inflight[c] = start_gather(c)
        pe_copy.wait()
        for c in range(n_chunks):
            b = c % _NBUF
            h = c % per_b
            inflight.pop(c).wait()

            @plsc.parallel_loop(0, C, unroll=2)
            def row_body(r):
                inv = jnp.float32(1.0 / _PE_SCALE)
                for g in range(D // (4 * _LANES)):
                    o = pl.multiple_of((h * C + r) * (D // 4) + g * _LANES, 8)
                    w = pe_res[pl.ds(o, _LANES)]
                    b8 = plsc.bitcast(w, jnp.int8)
                    l1a, l1b = plsc.unpack(
                        b8, format=plsc.PackFormat.INTERLEAVED,
                        preferred_element_type=jnp.int16)
                    f0, f1 = plsc.unpack(
                        l1a, format=plsc.PackFormat.INTERLEAVED,
                        preferred_element_type=jnp.int32)
                    f2, f3 = plsc.unpack(
                        l1b, format=plsc.PackFormat.INTERLEAVED,
                        preferred_element_type=jnp.int32)
                    base_col = g * 4 * _LANES
                    for k, fi in enumerate((f0, f1, f2, f3)):
                        plsc.addupdate(
                            rows_v.at[b, r, pl.ds(base_col + k * _LANES, _LANES)],
                            fi.astype(jnp.float32) * inv)

            off = (c // per_b) * S + p0 + h * C
            stores[c] = pltpu.async_copy(
                rows_v.at[b], out_hbm.at[pl.ds(off, C)], ssem[b])
            ahead = c + _NBUF - 1
            if ahead < n_chunks:
                if ahead - _NBUF >= 0:
                    stores[ahead - _NBUF].wait()
                inflight[ahead] = start_gather(ahead)
        for c in range(max(0, n_chunks - _NBUF), n_chunks):
            if c in stores:
                stores[c].wait()

    return emb_kernel


def kernel(x, table):
    B, S = x.shape
    _, D = table.shape
    pe = _packed_pe(S, D)
    out = _build_kernel(B, S, D, 32)(x.astype(jnp.int32), table, pe)
    return out.reshape(B, S, D)


# speedup vs baseline: 1.0581x; 1.0581x over previous
import functools

import ml_dtypes
import numpy as np
import jax
import jax.numpy as jnp
from jax import lax
from jax.experimental import pallas as pl
from jax.experimental.pallas import tpu as pltpu
from jax.experimental.pallas import tpu_sc as plsc

_MAX_LEN = 2048

_NUM_CORES = 2
_NUM_SUBCORES = 16
_NUM_WORKERS = _NUM_CORES * _NUM_SUBCORES
_LANES = 16
_NBUF = 4


def _positional_encoding(max_len, d_model):
    pos = np.arange(max_len, dtype=np.float32)[:, None]
    i2 = np.arange(0, d_model, 2, dtype=np.float32)
    div = np.power(10000.0, i2 / d_model)
    pe = np.zeros((max_len, d_model), dtype=np.float32)
    pe[:, 0::2] = np.sin(pos / div)
    pe[:, 1::2] = np.cos(pos / div)
    return pe


def _packed_pe(S, D):
    pe = _positional_encoding(_MAX_LEN, D)[:S]
    grp = pe.reshape(S, D // 32, 32)
    packed = np.empty_like(grp)
    packed[:, :, 0::2] = grp[:, :, :16]
    packed[:, :, 1::2] = grp[:, :, 16:]
    bf = packed.reshape(S * D).astype(ml_dtypes.bfloat16)
    return jnp.asarray(bf.view(np.int32))


@functools.cache
def _build_kernel(B, S, D, C):
    N = B * S
    W = S // _NUM_WORKERS
    n_chunks = B * W // C
    per_b = W // C
    mesh = plsc.VectorSubcoreMesh(core_axis_name="c", subcore_axis_name="s")

    @functools.partial(
        pl.kernel,
        out_type=jax.ShapeDtypeStruct((N, D), jnp.float32),
        mesh=mesh,
        compiler_params=pltpu.CompilerParams(needs_layout_passes=False),
        scratch_types=[
            pltpu.VMEM((B, W), jnp.int32),
            pltpu.VMEM((W * D // 2,), jnp.int32),
            pltpu.VMEM((_NBUF, C, D), jnp.float32),
            [pltpu.SemaphoreType.DMA] * _NBUF,
            [pltpu.SemaphoreType.DMA] * _NBUF,
            pltpu.SemaphoreType.DMA,
            pltpu.SemaphoreType.DMA,
        ],
    )
    def emb_kernel(x_hbm, table_hbm, pe_hbm, out_hbm,
                   idx_v, pe_res, rows_v, gsem, ssem, isem, psem):
        wid = lax.axis_index("s") * _NUM_CORES + lax.axis_index("c")
        p0 = wid * W

        pe_copy = pltpu.async_copy(
            pe_hbm.at[pl.ds(pl.multiple_of(p0 * (D // 2), 8), W * D // 2)],
            pe_res, psem)
        idx_copies = [
            pltpu.async_copy(x_hbm.at[blk, pl.ds(p0, W)], idx_v.at[blk], isem)
            for blk in range(B)
        ]
        idx_waited = [False] * B

        def start_gather(c):
            blk = c // per_b
            if not idx_waited[blk]:
                idx_copies[blk].wait()
                idx_waited[blk] = True
            b = c % _NBUF
            return pltpu.async_copy(
                table_hbm.at[idx_v.at[blk, pl.ds((c % per_b) * C, C)]],
                rows_v.at[b], gsem[b])

        inflight = {}
        stores = {}
        for c in range(min(_NBUF - 1, n_chunks)):
            inflight[c] = start_gather(c)
        pe_copy.wait()
        for c in range(n_chunks):
            b = c % _NBUF
            h = c % per_b
            inflight.pop(c).wait()

            @plsc.parallel_loop(0, C, unroll=2)
            def row_body(r):
                for g in range(D // (2 * _LANES)):
                    o = pl.multiple_of((h * C + r) * (D // 2) + g * _LANES, 8)
                    w = pe_res[pl.ds(o, _LANES)]
                    v = plsc.bitcast(w, jnp.bfloat16)
                    lo, hi = plsc.unpack(
                        v, format=plsc.PackFormat.INTERLEAVED,
                        preferred_element_type=jnp.float32)
                    plsc.addupdate(
                        rows_v.at[b, r, pl.ds(g * 2 * _LANES, _LANES)], lo)
                    plsc.addupdate(
                        rows_v.at[b, r, pl.ds(g * 2 * _LANES + _LANES, _LANES)], hi)

            off = (c // per_b) * S + p0 + h * C
            stores[c] = pltpu.async_copy(
                rows_v.at[b], out_hbm.at[pl.ds(off, C)], ssem[b])
            ahead = c + _NBUF - 1
            if ahead < n_chunks:
                if ahead - _NBUF >= 0:
                    stores[ahead - _NBUF].wait()
                inflight[ahead] = start_gather(ahead)
        for c in range(max(0, n_chunks - _NBUF), n_chunks):
            if c in stores:
                stores[c].wait()

    return emb_kernel


def kernel(x, table):
    B, S = x.shape
    _, D = table.shape
    pe = _packed_pe(S, D)
    out = _build_kernel(B, S, D, 32)(x.astype(jnp.int32), table, pe)
    return out.reshape(B, S, D)
